# Initial kernel scaffold; baseline (speedup 1.0000x reference)
import jax
import jax.numpy as jnp
from jax.experimental import pallas as pl


def kernel(x, W, b):
    raise NotImplementedError("write your pallas kernel here")


# speedup vs baseline: 5.2545x; 5.2545x over previous
import math

import jax
import jax.numpy as jnp
from jax.experimental import pallas as pl
from jax.experimental.pallas import tpu as pltpu

_B, _N, _C = 2, 4096, 4096
_E = 64
_K = 8
_CF = 1.25
_T = _B * _N
_BT = 512
_CAP = math.ceil(_CF * _T * _K / _E)


def _block_cumsum(c, bt, e):
    s = 1
    while s < bt:
        shifted = jnp.concatenate(
            [jnp.zeros((s, e), jnp.float32), c[: bt - s, :]], axis=0)
        c = c + shifted
        s *= 2
    return c


def _router_kernel(x_ref, wt_ref, b_ref, disp_ref, comb_ref, cnt_ref):
    i = pl.program_id(0)

    @pl.when(i == 0)
    def _init():
        cnt_ref[...] = jnp.zeros_like(cnt_ref)

    x = x_ref[...]
    logits = jnp.dot(x, wt_ref[...],
                     preferred_element_type=jnp.float32) + b_ref[...]
    m = jnp.max(logits, axis=1, keepdims=True)
    ex = jnp.exp(logits - m)
    probs = ex / jnp.sum(ex, axis=1, keepdims=True)

    iota = jax.lax.broadcasted_iota(jnp.int32, (_BT, _E), 1)

    p = probs
    onehots = []
    weights = []
    for _ in range(_K):
        mx = jnp.max(p, axis=1, keepdims=True)
        idx = jnp.min(jnp.where(p >= mx, iota, _E), axis=1, keepdims=True)
        oh = (iota == idx).astype(jnp.float32)
        onehots.append(oh)
        weights.append(mx)
        p = jnp.where(oh > 0, -1.0, p)

    wsum = weights[0]
    for r in range(1, _K):
        wsum = wsum + weights[r]
    denom = jnp.maximum(wsum, 1e-6)

    disp = jnp.zeros((_BT, _E), jnp.float32)
    comb = jnp.zeros((_BT, _E), jnp.float32)
    for r in range(_K):
        oh = onehots[r]
        c = _block_cumsum(oh, _BT, _E)
        pos = cnt_ref[r : r + 1, :] + c - 1.0
        acc = jnp.where((pos < _CAP) & (oh > 0), 1.0, 0.0)
        comb = comb + acc * (weights[r] / denom)
        disp = disp + acc
        cnt_ref[r : r + 1, :] += c[_BT - 1 : _BT, :]

    comb = comb * (disp > 0).astype(jnp.float32)
    comb = comb / jnp.maximum(jnp.sum(comb, axis=1, keepdims=True), 1e-6)
    disp_ref[...] = disp
    comb_ref[...] = comb


def kernel(x, W, b):
    xf = x.reshape(_T, _C)
    wt = W.T
    b2 = b.reshape(1, _E)
    disp, comb = pl.pallas_call(
        _router_kernel,
        grid=(_T // _BT,),
        in_specs=[
            pl.BlockSpec((_BT, _C), lambda i: (i, 0)),
            pl.BlockSpec((_C, _E), lambda i: (0, 0)),
            pl.BlockSpec((1, _E), lambda i: (0, 0)),
        ],
        out_specs=[
            pl.BlockSpec((_BT, _E), lambda i: (i, 0)),
            pl.BlockSpec((_BT, _E), lambda i: (i, 0)),
        ],
        out_shape=[
            jax.ShapeDtypeStruct((_T, _E), jnp.float32),
            jax.ShapeDtypeStruct((_T, _E), jnp.float32),
        ],
        scratch_shapes=[pltpu.VMEM((_K, _E), jnp.float32)],
        compiler_params=pltpu.CompilerParams(
            dimension_semantics=("arbitrary",),
        ),
    )(xf, wt, b2)
    return disp.reshape(_B, _N, _E), comb.reshape(_B, _N, _E)


# speedup vs baseline: 6.6676x; 1.2689x over previous
import math

import jax
import jax.numpy as jnp
from jax.experimental import pallas as pl
from jax.experimental.pallas import tpu as pltpu

_B, _N, _C = 2, 4096, 4096
_E = 64
_K = 8
_CF = 1.25
_T = _B * _N
_BT = 512
_CAP = math.ceil(_CF * _T * _K / _E)


def _block_cumsum(c, bt, e):
    s = 1
    while s < bt:
        shifted = jnp.concatenate(
            [jnp.zeros((s, e), jnp.float32), c[: bt - s, :]], axis=0)
        c = c + shifted
        s *= 2
    return c


def _router_kernel(x_ref, wt_ref, b_ref, disp_ref, comb_ref, cnt_ref):
    i = pl.program_id(0)

    @pl.when(i == 0)
    def _init():
        cnt_ref[...] = jnp.zeros_like(cnt_ref)

    x = x_ref[...]
    logits = jnp.dot(x, wt_ref[...],
                     preferred_element_type=jnp.float32) + b_ref[...]
    m = jnp.max(logits, axis=1, keepdims=True)
    ex = jnp.exp(logits - m)
    probs = ex / jnp.sum(ex, axis=1, keepdims=True)

    iota = jax.lax.broadcasted_iota(jnp.int32, (_BT, _E), 1)

    bits = jax.lax.bitcast_convert_type(probs, jnp.int32)
    v = jnp.bitwise_or(jnp.bitwise_and(bits, -64), (_E - 1) - iota)

    onehots = []
    weights = []
    for _ in range(_K):
        mxi = jnp.max(v, axis=1, keepdims=True)
        ohb = v == mxi
        onehots.append(ohb)
        weights.append(jax.lax.bitcast_convert_type(mxi, jnp.float32))
        v = jnp.where(ohb, -1, v)

    wsum = weights[0]
    for r in range(1, _K):
        wsum = wsum + weights[r]
    denom = jnp.maximum(wsum, 1e-6)

    colsums = jnp.concatenate(
        [jnp.sum(oh.astype(jnp.float32), axis=0, keepdims=True)
         for oh in onehots], axis=0)
    cnt_prev = cnt_ref[...]
    cnt_ref[...] = cnt_prev + colsums

    disp = jnp.zeros((_BT, _E), jnp.float32)
    comb = jnp.zeros((_BT, _E), jnp.float32)
    for r in range(_K):
        wr = weights[r] / denom
        comb = comb + jnp.where(onehots[r], wr, 0.0)
        disp = disp + jnp.where(onehots[r], 1.0, 0.0)
    comb = comb / jnp.maximum(jnp.sum(comb, axis=1, keepdims=True), 1e-6)
    disp_ref[...] = disp
    comb_ref[...] = comb

    @pl.when(jnp.max(cnt_prev + colsums) > _CAP)
    def _slow():
        disp = jnp.zeros((_BT, _E), jnp.float32)
        comb = jnp.zeros((_BT, _E), jnp.float32)
        for r in range(_K):
            oh = onehots[r].astype(jnp.float32)
            c = _block_cumsum(oh, _BT, _E)
            pos = cnt_prev[r : r + 1, :] + c - 1.0
            acc = jnp.where((pos < _CAP) & onehots[r], 1.0, 0.0)
            comb = comb + acc * (weights[r] / denom)
            disp = disp + acc
        comb = comb * (disp > 0).astype(jnp.float32)
        comb = comb / jnp.maximum(jnp.sum(comb, axis=1, keepdims=True), 1e-6)
        disp_ref[...] = disp
        comb_ref[...] = comb


def kernel(x, W, b):
    xf = x.reshape(_T, _C)
    wt = W.T
    b2 = b.reshape(1, _E)
    disp, comb = pl.pallas_call(
        _router_kernel,
        grid=(_T // _BT,),
        in_specs=[
            pl.BlockSpec((_BT, _C), lambda i: (i, 0)),
            pl.BlockSpec((_C, _E), lambda i: (0, 0)),
            pl.BlockSpec((1, _E), lambda i: (0, 0)),
        ],
        out_specs=[
            pl.BlockSpec((_BT, _E), lambda i: (i, 0)),
            pl.BlockSpec((_BT, _E), lambda i: (i, 0)),
        ],
        out_shape=[
            jax.ShapeDtypeStruct((_T, _E), jnp.float32),
            jax.ShapeDtypeStruct((_T, _E), jnp.float32),
        ],
        scratch_shapes=[pltpu.VMEM((_K, _E), jnp.float32)],
        compiler_params=pltpu.CompilerParams(
            dimension_semantics=("arbitrary",),
        ),
    )(xf, wt, b2)
    return disp.reshape(_B, _N, _E), comb.reshape(_B, _N, _E)


# speedup vs baseline: 6.9588x; 1.0437x over previous
import math

import jax
import jax.numpy as jnp
from jax.experimental import pallas as pl
from jax.experimental.pallas import tpu as pltpu

_B, _N, _C = 2, 4096, 4096
_E = 64
_K = 8
_CF = 1.25
_T = _B * _N
_BT = 512
_CAP = math.ceil(_CF * _T * _K / _E)


def _block_cumsum(c, bt, e):
    s = 1
    while s < bt:
        shifted = jnp.concatenate(
            [jnp.zeros((s, e), jnp.float32), c[: bt - s, :]], axis=0)
        c = c + shifted
        s *= 2
    return c


def _router_kernel(x_ref, wt_ref, b_ref, disp_ref, comb_ref, cnt_ref):
    i = pl.program_id(0)

    @pl.when(i == 0)
    def _init():
        cnt_ref[...] = jnp.zeros_like(cnt_ref)

    x = x_ref[...]
    logits = jnp.dot(x, wt_ref[...],
                     preferred_element_type=jnp.float32) + b_ref[...]
    m = jnp.max(logits, axis=1, keepdims=True)
    ex = jnp.exp(logits - m)

    iota = jax.lax.broadcasted_iota(jnp.int32, (_BT, _E), 1)
    bits = jax.lax.bitcast_convert_type(ex, jnp.int32)
    v0 = jnp.bitwise_or(jnp.bitwise_and(bits, -64), (_E - 1) - iota)

    v = v0
    csl = []
    for _ in range(_K):
        mxi = jnp.max(v, axis=1, keepdims=True)
        ohb = v == mxi
        csl.append(jnp.sum(jnp.where(ohb, 1.0, 0.0), axis=0, keepdims=True))
        v = jnp.where(ohb, -1, v)

    colsums = jnp.concatenate(csl, axis=0)
    cnt_prev = cnt_ref[...]
    cnt_ref[...] = cnt_prev + colsums

    support = v == -1
    disp = jnp.where(support, 1.0, 0.0)
    comb_raw = jnp.where(support, ex, 0.0)
    wsum = jnp.sum(comb_raw, axis=1, keepdims=True)
    denom = jnp.maximum(wsum, 1e-6)
    f = 1.0 / (denom * jnp.maximum(wsum / denom, 1e-6))
    disp_ref[...] = disp
    comb_ref[...] = comb_raw * f

    @pl.when(jnp.max(cnt_prev + colsums) > _CAP)
    def _slow():
        v = v0
        disp = jnp.zeros((_BT, _E), jnp.float32)
        comb = jnp.zeros((_BT, _E), jnp.float32)
        for r in range(_K):
            mxi = jnp.max(v, axis=1, keepdims=True)
            ohb = v == mxi
            oh = jnp.where(ohb, 1.0, 0.0)
            c = _block_cumsum(oh, _BT, _E)
            pos = cnt_prev[r : r + 1, :] + c - 1.0
            accb = (pos < _CAP) & ohb
            disp = disp + jnp.where(accb, 1.0, 0.0)
            comb = comb + jnp.where(accb, ex, 0.0)
            v = jnp.where(ohb, -1, v)
        support = v == -1
        wsum8 = jnp.sum(jnp.where(support, ex, 0.0), axis=1, keepdims=True)
        comb1 = comb / jnp.maximum(wsum8, 1e-6)
        rs = jnp.sum(comb1, axis=1, keepdims=True)
        comb1 = comb1 / jnp.maximum(rs, 1e-6)
        disp_ref[...] = disp
        comb_ref[...] = comb1


def kernel(x, W, b):
    xf = x.reshape(_T, _C)
    wt = W.T
    b2 = b.reshape(1, _E)
    disp, comb = pl.pallas_call(
        _router_kernel,
        grid=(_T // _BT,),
        in_specs=[
            pl.BlockSpec((_BT, _C), lambda i: (i, 0)),
            pl.BlockSpec((_C, _E), lambda i: (0, 0)),
            pl.BlockSpec((1, _E), lambda i: (0, 0)),
        ],
        out_specs=[
            pl.BlockSpec((_BT, _E), lambda i: (i, 0)),
            pl.BlockSpec((_BT, _E), lambda i: (i, 0)),
        ],
        out_shape=[
            jax.ShapeDtypeStruct((_T, _E), jnp.float32),
            jax.ShapeDtypeStruct((_T, _E), jnp.float32),
        ],
        scratch_shapes=[pltpu.VMEM((_K, _E), jnp.float32)],
        compiler_params=pltpu.CompilerParams(
            dimension_semantics=("arbitrary",),
        ),
    )(xf, wt, b2)
    return disp.reshape(_B, _N, _E), comb.reshape(_B, _N, _E)
